# baseline (speedup 1.0000x reference)
import functools

import jax
import jax.numpy as jnp
from jax import lax
from jax.experimental import pallas as pl
from jax.experimental.pallas import tpu as pltpu
from jax.experimental.pallas import tpu_sc as plsc

NC, NS, LANES = 2, 16, 16
NW = NC * NS

B, L = 16384, 50
D = 32
RPW = B // NW
CHUNKS = RPW // LANES
KROWS = (RPW * L) // 128


def _dot_body(e_ref, w_ref, o_ref):
    o_ref[...] = jnp.sum(e_ref[...] * w_ref[...], axis=0)


SCB = 65536


def _precompute_s(embed_t, w_col):
    V = embed_t.shape[1]
    grid = (V + SCB - 1) // SCB
    return pl.pallas_call(
        _dot_body,
        grid=(grid,),
        in_specs=[
            pl.BlockSpec((D, SCB), lambda i: (0, i)),
            pl.BlockSpec((D, 1), lambda i: (0, 0)),
        ],
        out_specs=pl.BlockSpec((SCB,), lambda i: (i,)),
        out_shape=jax.ShapeDtypeStruct((V,), jnp.float32),
    )(embed_t, w_col)


SBLK = 1024
LP = 64


def _sc_body(s_hbm, idx_hbm, mask_hbm, b_hbm, out_hbm,
             idx_v, mask_v, vals_v, out_v, b_v, sem):
    wid = lax.axis_index("s") * NC + lax.axis_index("c")
    base = wid * RPW
    copies = []
    for l in range(L):
        copies.append(pltpu.async_copy(
            idx_hbm.at[pl.ds(l * B + base, RPW)],
            idx_v.at[pl.ds(l * RPW, RPW)], sem))
        copies.append(pltpu.async_copy(
            mask_hbm.at[pl.ds(l * B + base, RPW)],
            mask_v.at[pl.ds(l * RPW, RPW)], sem))
    pltpu.sync_copy(b_hbm, b_v)
    for cp in copies:
        cp.wait()
    NSTAGE = 5
    HL = L // NSTAGE
    HN = HL * RPW
    gs = [pltpu.async_copy(
        s_hbm.at[idx_v.at[pl.ds(k * HN, HN)]],
        vals_v.at[pl.ds(k * HN, HN)], sem) for k in range(NSTAGE)]

    bias = b_v[...]
    zero = jnp.zeros((LANES,), jnp.float32)

    def stage(l_lo, l_hi, carries):
        res = []
        for c in range(CHUNKS):
            col = c * LANES
            def body(l, carry, col=col):
                acc, msum = carry
                off = l * RPW + col
                v = vals_v[pl.ds(off, LANES)]
                m = mask_v[pl.ds(off, LANES)]
                return acc + v * m, msum + m
            res.append(lax.fori_loop(l_lo, l_hi, body, carries[c]))
        return res

    part = [(zero, zero)] * CHUNKS
    for k in range(NSTAGE):
        gs[k].wait()
        part = stage(k * HL, (k + 1) * HL, part)
    for c in range(CHUNKS):
        acc, msum = part[c]
        out_v[pl.ds(c * LANES, LANES)] = acc / (msum + 1e-9) + bias
    pltpu.sync_copy(out_v, out_hbm.at[pl.ds(wid * RPW, RPW)])


@functools.cache
def _make_sc_call():
    mesh = plsc.VectorSubcoreMesh(
        core_axis_name="c", subcore_axis_name="s",
        num_cores=NC, num_subcores=NS)
    return pl.kernel(
        _sc_body,
        out_type=jax.ShapeDtypeStruct((B,), jnp.float32),
        mesh=mesh,
        compiler_params=pltpu.CompilerParams(needs_layout_passes=False),
        scratch_types=[
            pltpu.VMEM((L * RPW,), jnp.int32),
            pltpu.VMEM((L * RPW,), jnp.float32),
            pltpu.VMEM((L * RPW,), jnp.float32),
            pltpu.VMEM((RPW,), jnp.float32),
            pltpu.VMEM((LANES,), jnp.float32),
            pltpu.SemaphoreType.DMA,
        ],
    )


@jax.jit
def kernel(inputs, mask, embed, W, b):
    s = _precompute_s(embed.astype(jnp.float32).T,
                      W.astype(jnp.float32).reshape(D, 1))
    idx_t = inputs.astype(jnp.int32).T.reshape(L * B)
    mask_t = mask.astype(jnp.float32).T.reshape(L * B)
    b16 = jnp.broadcast_to(b.astype(jnp.float32).reshape(()), (LANES,))
    return _make_sc_call()(s, idx_t, mask_t, b16)


# speedup vs baseline: 1.1069x; 1.1069x over previous
import functools

import jax
import jax.numpy as jnp
from jax import lax
from jax.experimental import pallas as pl
from jax.experimental.pallas import tpu as pltpu
from jax.experimental.pallas import tpu_sc as plsc

NC, NS, LANES = 2, 16, 16
NW = NC * NS

B, L = 16384, 50
D = 32
RPW = B // NW
CHUNKS = RPW // LANES
KROWS = (RPW * L) // 128


def _dot_body(e_ref, w_ref, o_ref):
    o_ref[...] = jnp.sum(e_ref[...] * w_ref[...], axis=0)


SCB = 65536


def _precompute_s(embed_t, w_col):
    V = embed_t.shape[1]
    grid = (V + SCB - 1) // SCB
    return pl.pallas_call(
        _dot_body,
        grid=(grid,),
        in_specs=[
            pl.BlockSpec((D, SCB), lambda i: (0, i)),
            pl.BlockSpec((D, 1), lambda i: (0, 0)),
        ],
        out_specs=pl.BlockSpec((SCB,), lambda i: (i,)),
        out_shape=jax.ShapeDtypeStruct((V,), jnp.float32),
    )(embed_t, w_col)


SBLK = 1024
LP = 64


def _sc_body(s_hbm, idx_hbm, mask_hbm, b_hbm, out_hbm,
             idx_v, mask_v, vals_v, out_v, b_v, sem):
    wid = lax.axis_index("s") * NC + lax.axis_index("c")
    base = wid * RPW
    copies = []
    for l in range(L):
        copies.append(pltpu.async_copy(
            idx_hbm.at[pl.ds(l * B + base, RPW)],
            idx_v.at[pl.ds(l * RPW, RPW)], sem))
        copies.append(pltpu.async_copy(
            mask_hbm.at[pl.ds(l * B + base, RPW)],
            mask_v.at[pl.ds(l * RPW, RPW)], sem))
    pltpu.sync_copy(b_hbm, b_v)
    for cp in copies:
        cp.wait()
    NSTAGE = 2
    HL = L // NSTAGE
    HN = HL * RPW
    gs = [pltpu.async_copy(
        s_hbm.at[idx_v.at[pl.ds(k * HN, HN)]],
        vals_v.at[pl.ds(k * HN, HN)], sem) for k in range(NSTAGE)]

    bias = b_v[...]
    zero = jnp.zeros((LANES,), jnp.float32)

    def stage(l_lo, l_hi, carries):
        res = []
        for c in range(CHUNKS):
            col = c * LANES
            def body(l, carry, col=col):
                acc, msum = carry
                off = l * RPW + col
                v = vals_v[pl.ds(off, LANES)]
                m = mask_v[pl.ds(off, LANES)]
                return acc + v * m, msum + m
            res.append(lax.fori_loop(l_lo, l_hi, body, carries[c]))
        return res

    part = [(zero, zero)] * CHUNKS
    for k in range(NSTAGE):
        gs[k].wait()
        part = stage(k * HL, (k + 1) * HL, part)
    for c in range(CHUNKS):
        acc, msum = part[c]
        out_v[pl.ds(c * LANES, LANES)] = acc / (msum + 1e-9) + bias
    pltpu.sync_copy(out_v, out_hbm.at[pl.ds(wid * RPW, RPW)])


@functools.cache
def _make_sc_call():
    mesh = plsc.VectorSubcoreMesh(
        core_axis_name="c", subcore_axis_name="s",
        num_cores=NC, num_subcores=NS)
    return pl.kernel(
        _sc_body,
        out_type=jax.ShapeDtypeStruct((B,), jnp.float32),
        mesh=mesh,
        compiler_params=pltpu.CompilerParams(needs_layout_passes=False),
        scratch_types=[
            pltpu.VMEM((L * RPW,), jnp.int32),
            pltpu.VMEM((L * RPW,), jnp.float32),
            pltpu.VMEM((L * RPW,), jnp.float32),
            pltpu.VMEM((RPW,), jnp.float32),
            pltpu.VMEM((LANES,), jnp.float32),
            pltpu.SemaphoreType.DMA,
        ],
    )


@jax.jit
def kernel(inputs, mask, embed, W, b):
    s = _precompute_s(embed.astype(jnp.float32).T,
                      W.astype(jnp.float32).reshape(D, 1))
    idx_t = inputs.astype(jnp.int32).T.reshape(L * B)
    mask_t = mask.astype(jnp.float32).T.reshape(L * B)
    b16 = jnp.broadcast_to(b.astype(jnp.float32).reshape(()), (LANES,))
    return _make_sc_call()(s, idx_t, mask_t, b16)
